# baseline (speedup 1.0000x reference)
import functools

import jax
import jax.numpy as jnp
from jax import lax
from jax.experimental import pallas as pl
from jax.experimental.pallas import tpu as pltpu
from jax.experimental.pallas import tpu_sc as plsc

B = 16384
D = 64
NC = 2
NS = 16
NW = NC * NS
BPW = B // NW

NENT = 1000000
CW = 512
NCHUNKS = 999936 // CW
CPW = 61
LISTCAP = 8176
NROWS = 2 * B + 16

_mesh = plsc.VectorSubcoreMesh(core_axis_name="c", subcore_axis_name="s")


def _splat(x):
    return jnp.full((16,), x, jnp.int32)


@functools.partial(
    pl.kernel,
    mesh=_mesh,
    compiler_params=pltpu.CompilerParams(
        needs_layout_passes=False, disable_bounds_checks=True),
    out_type=jax.ShapeDtypeStruct((NROWS, 128), jnp.float32),
    scratch_types=[
        pltpu.VMEM((B,), jnp.int32),
        pltpu.VMEM((B,), jnp.int32),
        pltpu.VMEM((LISTCAP + 16,), jnp.int32),
        pltpu.VMEM((LISTCAP + 16,), jnp.int32),
        pltpu.VMEM((2, D, CW), jnp.float32),
        pltpu.VMEM((2, 16, 128), jnp.float32),
        pltpu.VMEM((2, 16), jnp.int32),
        pltpu.SemaphoreType.DMA,
        pltpu.SemaphoreType.DMA,
    ],
)
def _stream_extract_kernel(heads_hbm, tails_hbm, entT_hbm, tail_blk_hbm,
                           rows_hbm, h_all, t_all, ids_l, pos_l,
                           chunkbuf, rowblk, posbuf, sem_s, sem_w):
    wid = lax.axis_index("s") * NC + lax.axis_index("c")
    lanes = lax.iota(jnp.int32, 16)
    is_last = wid == NW - 1

    pltpu.sync_copy(heads_hbm, h_all)
    pltpu.sync_copy(tails_hbm, t_all)

    chunk0 = wid * CPW
    lo = chunk0 * CW
    hi = jnp.where(is_last, NENT, lo + CPW * CW)
    nchunks = jnp.where(is_last, CPW + 1, CPW)

    def scan_one(src, posoff):
        def body(v, n):
            ids16 = src[pl.ds(pl.multiple_of(v * 16, 16), 16)]
            m = (ids16 >= lo) & (ids16 < hi)
            pos16 = v * 16 + lanes + posoff
            plsc.store_compressed(ids_l.at[pl.ds(n, 16)], ids16, mask=m)
            plsc.store_compressed(pos_l.at[pl.ds(n, 16)], pos16, mask=m)
            cnt = plsc.all_reduce_population_count(m)[0]
            return jnp.minimum(n + cnt, LISTCAP)
        return body

    n = lax.fori_loop(0, B // 16, scan_one(h_all, 0), 0)
    n = lax.fori_loop(0, B // 16, scan_one(t_all, B), n)
    nvecs = (n + 15) >> 4

    def extract(p, chunk_lo, width):
        def vec_body(v, carry):
            vp = v & 1
            off = pl.ds(pl.multiple_of(v * 16, 16), 16)
            ids16 = ids_l[off]
            pos16 = pos_l[off]
            rel = ids16 - chunk_lo
            m = (rel >= 0) & (rel < width)
            cols = jnp.clip(rel, 0, width - 1)
            pos_eff = jnp.where(m, pos16, 2 * B + lanes)

            @pl.when(v >= 2)
            def _():
                pltpu.make_async_copy(
                    rowblk.at[0], rows_hbm.at[posbuf.at[0]], sem_w).wait()

            posbuf[vp, :] = pos_eff
            for e in range(16):
                ce = cols[e]
                for c in range(D // 16):
                    vals = plsc.load_gather(
                        chunkbuf, [_splat(p), c * 16 + lanes, _splat(ce)])
                    rowblk[vp, e, pl.ds(c * 16, 16)] = vals
            pltpu.async_copy(rowblk.at[vp], rows_hbm.at[posbuf.at[vp]],
                             sem_w)
            return carry

        lax.fori_loop(0, nvecs, vec_body, 0)
        for k in range(2):
            @pl.when(nvecs >= k + 1)
            def _():
                pltpu.make_async_copy(
                    rowblk.at[0], rows_hbm.at[posbuf.at[0]], sem_w).wait()

    def start_chunk(c, p):
        src = entT_hbm.at[:, pl.ds(pl.multiple_of(c * CW, CW), CW)]
        pltpu.async_copy(src, chunkbuf.at[p], sem_s)

    def wait_chunk(p):
        pltpu.make_async_copy(
            entT_hbm.at[:, pl.ds(0, CW)], chunkbuf.at[p], sem_s).wait()

    start_chunk(chunk0, 0)

    def pair_body(pair, carry):
        for b in range(2):
            lc = pair * 2 + b

            @pl.when(lc < CPW)
            def _():
                wait_chunk(b)

                @pl.when(lc + 1 < CPW)
                def _():
                    start_chunk(chunk0 + lc + 1, 1 - b)
                extract(b, (chunk0 + lc) * CW, CW)
        return carry

    lax.fori_loop(0, (CPW + 1) // 2, pair_body, 0)

    @pl.when(is_last)
    def _():
        start_chunk(chunk0 + CPW, 1)
        wait_chunk(1)
        extract(1, (chunk0 + CPW) * CW, CW)
        pltpu.sync_copy(tail_blk_hbm, chunkbuf.at[0, :, pl.ds(0, 128)])
        extract(0, 999872, 128)


@functools.partial(
    pl.kernel,
    mesh=_mesh,
    compiler_params=pltpu.CompilerParams(
        needs_layout_passes=False, use_tc_tiling_on_sc=False,
        disable_bounds_checks=True),
    out_type=jax.ShapeDtypeStruct((B,), jnp.float32),
    scratch_types=[
        pltpu.VMEM((BPW,), jnp.int32),
        pltpu.VMEM((BPW, D), jnp.float32),
        pltpu.VMEM((BPW, D), jnp.float32),
        pltpu.VMEM((BPW, D), jnp.float32),
        pltpu.VMEM((BPW,), jnp.float32),
        pltpu.VMEM((256,), jnp.float32),
        pltpu.SemaphoreType.DMA,
    ],
)
def _score_kernel(rel_hbm, relt_hbm, rows_hbm, out_hbm,
                  r_idx, h_rows, r_rows, t_rows, out_v, tbuf, sem):
    wid = lax.axis_index("s") * NC + lax.axis_index("c")
    base = wid * BPW

    pltpu.sync_copy(rel_hbm.at[pl.ds(base, BPW)], r_idx)
    copies = [
        pltpu.async_copy(
            rows_hbm.at[pl.ds(base, BPW), pl.ds(0, D)], h_rows, sem),
        pltpu.async_copy(
            rows_hbm.at[pl.ds(B + base, BPW), pl.ds(0, D)], t_rows, sem),
    ]
    for c in range(BPW // 128):
        sl = pl.ds(c * 128, 128)
        copies.append(
            pltpu.async_copy(relt_hbm.at[r_idx.at[sl]], r_rows.at[sl], sem))
    for cp in copies:
        cp.wait()

    lanes = lax.iota(jnp.int32, 16)
    colbase = lanes * 16

    def group_body(g, carry):
        for e in range(16):
            b = g * 16 + e
            for c in range(D // 16):
                sl = pl.ds(c * 16, 16)
                d = (h_rows[b, sl] + r_rows[b, sl]) - t_rows[b, sl]
                if c == 0:
                    acc = d * d
                else:
                    acc = acc + d * d
            tbuf[pl.ds(e * 16, 16)] = acc
        tot = jnp.zeros((16,), jnp.float32)
        for k in range(16):
            tot = tot + plsc.load_gather(tbuf, [colbase + k])
        x = tot + 2e-38
        xi = plsc.bitcast(x, jnp.int32)
        y = plsc.bitcast((xi >> 1) + 0x1FBD1DF5, jnp.float32)
        y = 0.5 * (y + x / y)
        y = 0.5 * (y + x / y)
        out_v[pl.ds(pl.multiple_of(g * 16, 16), 16)] = -y
        return carry

    lax.fori_loop(0, BPW // 16, group_body, 0)
    pltpu.sync_copy(out_v, out_hbm.at[pl.ds(base, BPW)])


def kernel(heads, relations, tails, entity_table, relation_table):
    entT = entity_table.T
    tail_blk = lax.optimization_barrier(entT[:, 999872:])
    rel_lin = lax.optimization_barrier(relation_table.T).T
    rows = _stream_extract_kernel(heads, tails, entT, tail_blk)
    return _score_kernel(relations, rel_lin, rows)


# speedup vs baseline: 12.0325x; 12.0325x over previous
import functools

import jax
import jax.numpy as jnp
from jax import lax
from jax.experimental import pallas as pl
from jax.experimental.pallas import tpu as pltpu
from jax.experimental.pallas import tpu_sc as plsc

B = 16384
D = 64
NC = 2
NS = 16
NW = NC * NS
BPW = B // NW

NENT = 1000000
CW = 256
CPW = 122
RANGE = CPW * CW
NG = 8
FCAP = 4080
SCAP = 2032
HCAP = 496
NROWS = 2 * B + 16

_mesh = plsc.VectorSubcoreMesh(core_axis_name="c", subcore_axis_name="s")


def _splat(x):
    return jnp.full((16,), x, jnp.int32)


@functools.partial(
    pl.kernel,
    mesh=_mesh,
    compiler_params=pltpu.CompilerParams(
        needs_layout_passes=False, disable_bounds_checks=True),
    out_type=jax.ShapeDtypeStruct((NROWS, 128), jnp.float32),
    scratch_types=[
        pltpu.VMEM((B,), jnp.int32),
        pltpu.VMEM((B,), jnp.int32),
        pltpu.VMEM((FCAP + 16,), jnp.int32),
        pltpu.VMEM((FCAP + 16,), jnp.int32),
        pltpu.VMEM((NG * (SCAP + 16),), jnp.int32),
        pltpu.VMEM((NG * (SCAP + 16),), jnp.int32),
        pltpu.VMEM((HCAP + 16,), jnp.int32),
        pltpu.VMEM((HCAP + 16,), jnp.int32),
        pltpu.VMEM((2, D, CW), jnp.float32),
        pltpu.VMEM((2, 16, 128), jnp.float32),
        pltpu.VMEM((2, 16), jnp.int32),
        pltpu.SemaphoreType.DMA,
        pltpu.SemaphoreType.DMA,
    ],
)
def _stream_extract_kernel(heads_hbm, tails_hbm, entT_hbm, tail_blk_hbm,
                           rows_hbm, h_all, t_all, fids, fpos, gids, gpos,
                           hcols, hpos, chunkbuf, rowblk, posbuf,
                           sem_s, sem_w):
    wid = lax.axis_index("s") * NC + lax.axis_index("c")
    lanes = lax.iota(jnp.int32, 16)
    is_last = wid == NW - 1

    pltpu.sync_copy(heads_hbm, h_all)
    pltpu.sync_copy(tails_hbm, t_all)

    chunk0 = wid * CPW
    lo = chunk0 * CW
    hi = jnp.where(is_last, NENT, lo + RANGE)
    nchunks = jnp.where(is_last, CPW + 2, CPW)

    def scan_one(src, posoff):
        def body(v, n):
            ids16 = src[pl.ds(pl.multiple_of(v * 16, 16), 16)]
            m = (ids16 >= lo) & (ids16 < hi)
            pos16 = v * 16 + lanes + posoff
            plsc.store_compressed(fids.at[pl.ds(n, 16)], ids16, mask=m)
            plsc.store_compressed(fpos.at[pl.ds(n, 16)], pos16, mask=m)
            cnt = plsc.all_reduce_population_count(m)[0]
            return jnp.minimum(n + cnt, FCAP)
        return body

    n = lax.fori_loop(0, B // 16, scan_one(h_all, 0), 0)
    n = lax.fori_loop(0, B // 16, scan_one(t_all, B), n)
    nvecs = (n + 15) >> 4

    n_g = []
    for g in range(NG):
        gbase = g * (SCAP + 16)

        def dist_body(v, ng, g=g, gbase=gbase):
            off = pl.ds(pl.multiple_of(v * 16, 16), 16)
            ids16 = fids[off]
            pos16 = fpos[off]
            rel = ids16 - lo
            m = ((rel >= g * 4096) & (rel < (g + 1) * 4096)
                 & (v * 16 + lanes < n))
            plsc.store_compressed(gids.at[pl.ds(gbase + ng, 16)], ids16,
                                  mask=m)
            plsc.store_compressed(gpos.at[pl.ds(gbase + ng, 16)], pos16,
                                  mask=m)
            cnt = plsc.all_reduce_population_count(m)[0]
            return jnp.minimum(ng + cnt, SCAP)

        n_g.append(lax.fori_loop(0, nvecs, dist_body, 0))

    def extract(p, chunk_lo, width, g, ng):
        gbase = g * (SCAP + 16)
        gvecs = (ng + 15) >> 4

        def comp_body(v, nh):
            off = pl.ds(pl.multiple_of(gbase + v * 16, 16), 16)
            ids16 = gids[off]
            pos16 = gpos[off]
            rel = ids16 - chunk_lo
            m = (rel >= 0) & (rel < width) & (v * 16 + lanes < ng)
            plsc.store_compressed(hcols.at[pl.ds(nh, 16)], rel, mask=m)
            plsc.store_compressed(hpos.at[pl.ds(nh, 16)], pos16, mask=m)
            cnt = plsc.all_reduce_population_count(m)[0]
            return jnp.minimum(nh + cnt, HCAP)

        nh = lax.fori_loop(0, gvecs, comp_body, 0)
        nhv = (nh + 15) >> 4

        def hit_body(v, carry):
            vp = v & 1
            off = pl.ds(pl.multiple_of(v * 16, 16), 16)
            cols = jnp.clip(hcols[off], 0, width - 1)
            pos16 = hpos[off]
            valid = v * 16 + lanes < nh
            pos_eff = jnp.where(valid, pos16, 2 * B + lanes)

            @pl.when(v >= 2)
            def _():
                pltpu.make_async_copy(
                    rowblk.at[0], rows_hbm.at[posbuf.at[0]], sem_w).wait()

            posbuf[vp, :] = pos_eff
            for e in range(16):
                ce = cols[e]
                for c in range(D // 16):
                    vals = plsc.load_gather(
                        chunkbuf, [_splat(p), c * 16 + lanes, _splat(ce)])
                    rowblk[vp, e, pl.ds(c * 16, 16)] = vals
            pltpu.async_copy(rowblk.at[vp], rows_hbm.at[posbuf.at[vp]],
                             sem_w)
            return carry

        lax.fori_loop(0, nhv, hit_body, 0)
        for k in range(2):
            @pl.when(nhv >= k + 1)
            def _():
                pltpu.make_async_copy(
                    rowblk.at[0], rows_hbm.at[posbuf.at[0]], sem_w).wait()

    def start_chunk(c, p):
        src = entT_hbm.at[:, pl.ds(pl.multiple_of(c * CW, CW), CW)]
        pltpu.async_copy(src, chunkbuf.at[p], sem_s)

    def wait_chunk(p):
        pltpu.make_async_copy(
            entT_hbm.at[:, pl.ds(0, CW)], chunkbuf.at[p], sem_s).wait()

    start_chunk(chunk0, 0)

    def chunk_body(lc, carry):
        p = lc & 1
        g = lc >> 4
        ngv = n_g[0]
        for g_s in range(1, NG):
            ngv = jnp.where(g == g_s, n_g[g_s], ngv)

        wait_chunk(p)

        @pl.when(lc + 1 < nchunks)
        def _():
            start_chunk(chunk0 + lc + 1, 1 - p)
        extract(p, lo + lc * CW, CW, g, ngv)
        return carry

    lax.fori_loop(0, nchunks, chunk_body, 0)

    @pl.when(is_last)
    def _():
        pltpu.sync_copy(tail_blk_hbm, chunkbuf.at[0, :, pl.ds(0, 128)])
        extract(0, 999872, 128, NG - 1, n_g[NG - 1])


@functools.partial(
    pl.kernel,
    mesh=_mesh,
    compiler_params=pltpu.CompilerParams(
        needs_layout_passes=False, use_tc_tiling_on_sc=False,
        disable_bounds_checks=True),
    out_type=jax.ShapeDtypeStruct((B,), jnp.float32),
    scratch_types=[
        pltpu.VMEM((BPW,), jnp.int32),
        pltpu.VMEM((BPW, D), jnp.float32),
        pltpu.VMEM((BPW, D), jnp.float32),
        pltpu.VMEM((BPW, D), jnp.float32),
        pltpu.VMEM((BPW,), jnp.float32),
        pltpu.VMEM((256,), jnp.float32),
        pltpu.SemaphoreType.DMA,
    ],
)
def _score_kernel(rel_hbm, relt_hbm, rows_hbm, out_hbm,
                  r_idx, h_rows, r_rows, t_rows, out_v, tbuf, sem):
    wid = lax.axis_index("s") * NC + lax.axis_index("c")
    base = wid * BPW

    pltpu.sync_copy(rel_hbm.at[pl.ds(base, BPW)], r_idx)
    copies = [
        pltpu.async_copy(
            rows_hbm.at[pl.ds(base, BPW), pl.ds(0, D)], h_rows, sem),
        pltpu.async_copy(
            rows_hbm.at[pl.ds(B + base, BPW), pl.ds(0, D)], t_rows, sem),
    ]
    for c in range(BPW // 128):
        sl = pl.ds(c * 128, 128)
        copies.append(
            pltpu.async_copy(relt_hbm.at[r_idx.at[sl]], r_rows.at[sl], sem))
    for cp in copies:
        cp.wait()

    lanes = lax.iota(jnp.int32, 16)
    colbase = lanes * 16

    def group_body(g, carry):
        for e in range(16):
            b = g * 16 + e
            for c in range(D // 16):
                sl = pl.ds(c * 16, 16)
                d = (h_rows[b, sl] + r_rows[b, sl]) - t_rows[b, sl]
                if c == 0:
                    acc = d * d
                else:
                    acc = acc + d * d
            tbuf[pl.ds(e * 16, 16)] = acc
        tot = jnp.zeros((16,), jnp.float32)
        for k in range(16):
            tot = tot + plsc.load_gather(tbuf, [colbase + k])
        x = tot + 2e-38
        xi = plsc.bitcast(x, jnp.int32)
        y = plsc.bitcast((xi >> 1) + 0x1FBD1DF5, jnp.float32)
        y = 0.5 * (y + x / y)
        y = 0.5 * (y + x / y)
        out_v[pl.ds(pl.multiple_of(g * 16, 16), 16)] = -y
        return carry

    lax.fori_loop(0, BPW // 16, group_body, 0)
    pltpu.sync_copy(out_v, out_hbm.at[pl.ds(base, BPW)])


def kernel(heads, relations, tails, entity_table, relation_table):
    entT = entity_table.T
    tail_blk = lax.optimization_barrier(entT[:, 999872:])
    rel_lin = lax.optimization_barrier(relation_table.T).T
    rows = _stream_extract_kernel(heads, tails, entT, tail_blk)
    return _score_kernel(relations, rel_lin, rows)


# speedup vs baseline: 12.2810x; 1.0207x over previous
import functools

import jax
import jax.numpy as jnp
from jax import lax
from jax.experimental import pallas as pl
from jax.experimental.pallas import tpu as pltpu
from jax.experimental.pallas import tpu_sc as plsc

B = 16384
D = 64
NC = 2
NS = 16
NW = NC * NS
BPW = B // NW

NENT = 1000000
CW = 256
CPW = 122
RANGE = CPW * CW
NG = 8
FCAP = 4080
SCAP = 2032
HCAP = 496
NROWS = 2 * B + 16

_mesh = plsc.VectorSubcoreMesh(core_axis_name="c", subcore_axis_name="s")


def _splat(x):
    return jnp.full((16,), x, jnp.int32)


@functools.partial(
    pl.kernel,
    mesh=_mesh,
    compiler_params=pltpu.CompilerParams(
        needs_layout_passes=False, disable_bounds_checks=True),
    out_type=jax.ShapeDtypeStruct((NROWS, 128), jnp.float32),
    scratch_types=[
        pltpu.VMEM((B,), jnp.int32),
        pltpu.VMEM((B,), jnp.int32),
        pltpu.VMEM((FCAP + 16,), jnp.int32),
        pltpu.VMEM((FCAP + 16,), jnp.int32),
        pltpu.VMEM((NG * (SCAP + 16),), jnp.int32),
        pltpu.VMEM((NG * (SCAP + 16),), jnp.int32),
        pltpu.VMEM((HCAP + 16,), jnp.int32),
        pltpu.VMEM((HCAP + 16,), jnp.int32),
        pltpu.VMEM((2, D, CW), jnp.float32),
        pltpu.VMEM((2, 16, 128), jnp.float32),
        pltpu.VMEM((2, 16), jnp.int32),
        pltpu.SemaphoreType.DMA,
        pltpu.SemaphoreType.DMA,
    ],
)
def _stream_extract_kernel(heads_hbm, tails_hbm, entT_hbm, tail_blk_hbm,
                           rows_hbm, h_all, t_all, fids, fpos, gids, gpos,
                           hcols, hpos, chunkbuf, rowblk, posbuf,
                           sem_s, sem_w):
    wid = lax.axis_index("s") * NC + lax.axis_index("c")
    lanes = lax.iota(jnp.int32, 16)
    is_last = wid == NW - 1

    chunk0 = wid * CPW
    lo = chunk0 * CW
    hi = jnp.where(is_last, NENT, lo + RANGE)
    nchunks = jnp.where(is_last, CPW + 2, CPW)

    def start_chunk(c, p):
        src = entT_hbm.at[:, pl.ds(pl.multiple_of(c * CW, CW), CW)]
        pltpu.async_copy(src, chunkbuf.at[p], sem_s)

    def wait_chunk(p):
        pltpu.make_async_copy(
            entT_hbm.at[:, pl.ds(0, CW)], chunkbuf.at[p], sem_s).wait()

    start_chunk(chunk0, 0)
    start_chunk(chunk0 + 1, 1)
    pltpu.sync_copy(heads_hbm, h_all)
    pltpu.sync_copy(tails_hbm, t_all)

    HSEG = 2048

    def scan_body(v, carry):
        nh, nt = carry
        off = pl.ds(pl.multiple_of(v * 16, 16), 16)
        hv = h_all[off]
        tv = t_all[off]
        mh = (hv >= lo) & (hv < hi)
        mt = (tv >= lo) & (tv < hi)
        pos16 = v * 16 + lanes
        plsc.store_compressed(fids.at[pl.ds(nh, 16)], hv, mask=mh)
        plsc.store_compressed(fids.at[pl.ds(HSEG + nt, 16)], tv, mask=mt)
        plsc.store_compressed(fpos.at[pl.ds(nh, 16)], pos16, mask=mh)
        plsc.store_compressed(fpos.at[pl.ds(HSEG + nt, 16)], pos16 + B,
                              mask=mt)
        ch = plsc.all_reduce_population_count(mh)[0]
        ct = plsc.all_reduce_population_count(mt)[0]
        return (jnp.minimum(nh + ch, HSEG - 16),
                jnp.minimum(nt + ct, HSEG - 16))

    n_h, n_t = lax.fori_loop(0, B // 16, scan_body, (0, 0))
    nvecs_h = (n_h + 15) >> 4
    nvecs_t = (n_t + 15) >> 4

    n_g = []
    for g in range(NG):
        gbase = g * (SCAP + 16)

        def dist_body(seg_base, seg_n):
            def body(v, ng, g=g, gbase=gbase):
                off = pl.ds(pl.multiple_of(seg_base + v * 16, 16), 16)
                ids16 = fids[off]
                pos16 = fpos[off]
                rel = ids16 - lo
                m = ((rel >= g * 4096) & (rel < (g + 1) * 4096)
                     & (v * 16 + lanes < seg_n))
                plsc.store_compressed(gids.at[pl.ds(gbase + ng, 16)],
                                      ids16, mask=m)
                plsc.store_compressed(gpos.at[pl.ds(gbase + ng, 16)],
                                      pos16, mask=m)
                cnt = plsc.all_reduce_population_count(m)[0]
                return jnp.minimum(ng + cnt, SCAP)
            return body

        ng = lax.fori_loop(0, nvecs_h, dist_body(0, n_h), 0)
        ng = lax.fori_loop(0, nvecs_t, dist_body(HSEG, n_t), ng)
        n_g.append(ng)

    def extract(p, chunk_lo, width, g, ng):
        gbase = g * (SCAP + 16)
        gvecs = (ng + 15) >> 4

        def comp_body(v, nh):
            off = pl.ds(pl.multiple_of(gbase + v * 16, 16), 16)
            ids16 = gids[off]
            pos16 = gpos[off]
            rel = ids16 - chunk_lo
            m = (rel >= 0) & (rel < width) & (v * 16 + lanes < ng)
            plsc.store_compressed(hcols.at[pl.ds(nh, 16)], rel, mask=m)
            plsc.store_compressed(hpos.at[pl.ds(nh, 16)], pos16, mask=m)
            cnt = plsc.all_reduce_population_count(m)[0]
            return jnp.minimum(nh + cnt, HCAP)

        nh = lax.fori_loop(0, gvecs, comp_body, 0)
        nhv = (nh + 15) >> 4

        def hit_body(v, carry):
            vp = v & 1
            off = pl.ds(pl.multiple_of(v * 16, 16), 16)
            cols = jnp.clip(hcols[off], 0, width - 1)
            pos16 = hpos[off]
            valid = v * 16 + lanes < nh
            pos_eff = jnp.where(valid, pos16, 2 * B + lanes)

            @pl.when(v >= 2)
            def _():
                pltpu.make_async_copy(
                    rowblk.at[0], rows_hbm.at[posbuf.at[0]], sem_w).wait()

            posbuf[vp, :] = pos_eff
            for e in range(16):
                ce = cols[e]
                for c in range(D // 16):
                    vals = plsc.load_gather(
                        chunkbuf, [_splat(p), c * 16 + lanes, _splat(ce)])
                    rowblk[vp, e, pl.ds(c * 16, 16)] = vals
            pltpu.async_copy(rowblk.at[vp], rows_hbm.at[posbuf.at[vp]],
                             sem_w)
            return carry

        lax.fori_loop(0, nhv, hit_body, 0)
        for k in range(2):
            @pl.when(nhv >= k + 1)
            def _():
                pltpu.make_async_copy(
                    rowblk.at[0], rows_hbm.at[posbuf.at[0]], sem_w).wait()

    def chunk_body(lc, carry):
        p = lc & 1
        g = lc >> 4
        ngv = n_g[0]
        for g_s in range(1, NG):
            ngv = jnp.where(g == g_s, n_g[g_s], ngv)

        wait_chunk(p)
        extract(p, lo + lc * CW, CW, g, ngv)

        @pl.when(lc + 2 < nchunks)
        def _():
            start_chunk(chunk0 + lc + 2, p)
        return carry

    lax.fori_loop(0, nchunks, chunk_body, 0)

    @pl.when(is_last)
    def _():
        pltpu.sync_copy(tail_blk_hbm, chunkbuf.at[0, :, pl.ds(0, 128)])
        extract(0, 999872, 128, NG - 1, n_g[NG - 1])


@functools.partial(
    pl.kernel,
    mesh=_mesh,
    compiler_params=pltpu.CompilerParams(
        needs_layout_passes=False, use_tc_tiling_on_sc=False,
        disable_bounds_checks=True),
    out_type=jax.ShapeDtypeStruct((B,), jnp.float32),
    scratch_types=[
        pltpu.VMEM((BPW,), jnp.int32),
        pltpu.VMEM((BPW, D), jnp.float32),
        pltpu.VMEM((BPW, D), jnp.float32),
        pltpu.VMEM((BPW, D), jnp.float32),
        pltpu.VMEM((BPW,), jnp.float32),
        pltpu.VMEM((256,), jnp.float32),
        pltpu.SemaphoreType.DMA,
    ],
)
def _score_kernel(rel_hbm, relt_hbm, rows_hbm, out_hbm,
                  r_idx, h_rows, r_rows, t_rows, out_v, tbuf, sem):
    wid = lax.axis_index("s") * NC + lax.axis_index("c")
    base = wid * BPW

    pltpu.sync_copy(rel_hbm.at[pl.ds(base, BPW)], r_idx)
    copies = [
        pltpu.async_copy(
            rows_hbm.at[pl.ds(base, BPW), pl.ds(0, D)], h_rows, sem),
        pltpu.async_copy(
            rows_hbm.at[pl.ds(B + base, BPW), pl.ds(0, D)], t_rows, sem),
    ]
    for c in range(BPW // 128):
        sl = pl.ds(c * 128, 128)
        copies.append(
            pltpu.async_copy(relt_hbm.at[r_idx.at[sl]], r_rows.at[sl], sem))
    for cp in copies:
        cp.wait()

    lanes = lax.iota(jnp.int32, 16)
    colbase = lanes * 16

    def group_body(g, carry):
        for e in range(16):
            b = g * 16 + e
            for c in range(D // 16):
                sl = pl.ds(c * 16, 16)
                d = (h_rows[b, sl] + r_rows[b, sl]) - t_rows[b, sl]
                if c == 0:
                    acc = d * d
                else:
                    acc = acc + d * d
            tbuf[pl.ds(e * 16, 16)] = acc
        tot = jnp.zeros((16,), jnp.float32)
        for k in range(16):
            tot = tot + plsc.load_gather(tbuf, [colbase + k])
        x = tot + 2e-38
        xi = plsc.bitcast(x, jnp.int32)
        y = plsc.bitcast((xi >> 1) + 0x1FBD1DF5, jnp.float32)
        y = 0.5 * (y + x / y)
        y = 0.5 * (y + x / y)
        out_v[pl.ds(pl.multiple_of(g * 16, 16), 16)] = -y
        return carry

    lax.fori_loop(0, BPW // 16, group_body, 0)
    pltpu.sync_copy(out_v, out_hbm.at[pl.ds(base, BPW)])


def kernel(heads, relations, tails, entity_table, relation_table):
    entT = entity_table.T
    tail_blk = lax.optimization_barrier(entT[:, 999872:])
    rel_lin = lax.optimization_barrier(relation_table.T).T
    rows = _stream_extract_kernel(heads, tails, entT, tail_blk)
    return _score_kernel(relations, rel_lin, rows)
